# baseline (speedup 1.0000x reference)
import functools

import jax
import jax.numpy as jnp
from jax import lax
from jax.experimental import pallas as pl
from jax.experimental.pallas import tpu as pltpu
from jax.experimental.pallas import tpu_sc as plsc

MODEL_DIM = 128
MAX_LEN = 200
BATCH = 4096

NC = 2
NS = 16
NW = NC * NS
RPW = BATCH // NW
ZROWS = 192

_mesh = plsc.VectorSubcoreMesh(
    core_axis_name="c", subcore_axis_name="s", num_cores=NC, num_subcores=NS)


@functools.partial(
    pl.kernel,
    out_type=(
        jax.ShapeDtypeStruct((BATCH, MAX_LEN, MODEL_DIM), jnp.float32),
        jax.ShapeDtypeStruct((BATCH * MAX_LEN,), jnp.int32),
    ),
    mesh=_mesh,
    compiler_params=pltpu.CompilerParams(use_tc_tiling_on_sc=False),
    scratch_types=[
        pltpu.VMEM((MAX_LEN, MODEL_DIM), jnp.float32),
        pltpu.VMEM((ZROWS, MODEL_DIM), jnp.float32),
        pltpu.VMEM((RPW,), jnp.int32),
        pltpu.VMEM((RPW * MAX_LEN,), jnp.int32),
        pltpu.SemaphoreType.DMA,
    ],
)
def _pe_kernel(len_hbm, table_hbm, emb_hbm, pos_hbm,
               tbl_v, zero_v, lens_v, pos_v, sem):
    wid = lax.axis_index("s") * NC + lax.axis_index("c")
    base = wid * RPW

    pltpu.sync_copy(table_hbm.at[pl.ds(1, MAX_LEN)], tbl_v)
    pltpu.sync_copy(len_hbm.at[pl.ds(base, RPW)], lens_v)

    zvec = jnp.zeros((16,), jnp.float32)

    def _zero_row(r, _):
        for c in range(MODEL_DIM // 16):
            zero_v[r, pl.ds(c * 16, 16)] = zvec
        return 0

    lax.fori_loop(0, ZROWS, _zero_row, 0)

    iota = lax.iota(jnp.int32, 16)

    def _copy_digits(from_table, row, start, n):
        for scale, dmax, m in ((64, 3, 0), (8, 7, -64), (1, 7, -8)):
            digit = (n // scale) & 7
            rel = n & m
            for k in range(1, dmax + 1):
                size = scale * k

                @pl.when(digit == k)
                def _():
                    if from_table:
                        src = tbl_v.at[pl.ds(start + rel, size)]
                    else:
                        src = zero_v.at[pl.ds(0, size)]
                    pltpu.async_copy(
                        src, emb_hbm.at[row, pl.ds(start + rel, size)], sem)

    def _blk(blk, _):
        lv = lens_v[pl.ds(blk * 16, 16)]
        for lane in range(16):
            L = lv[lane]
            r = blk * 16 + lane
            row = base + r
            _copy_digits(True, row, 0, L)
            _copy_digits(False, row, L, MAX_LEN - L)
        for p in range(8):
            la = lv[2 * p]
            lb = lv[2 * p + 1]
            sa = jnp.full((16,), la, jnp.int32)
            sb = jnp.full((16,), lb, jnp.int32)
            fbase = (blk * 16 + 2 * p) * MAX_LEN
            for c in range(25):
                if c <= 11:
                    vec = iota + (16 * c + 1)
                    splat = sa
                elif c == 12:
                    vec = jnp.where(iota < 8, iota + 193, iota - 7)
                    splat = jnp.where(iota < 8, sa, sb)
                else:
                    vec = iota + (16 * c - 200 + 1)
                    splat = sb
                pos_v[pl.ds(fbase + 16 * c, 16)] = jnp.where(
                    vec <= splat, vec, 0)
        return 0

    lax.fori_loop(0, RPW // 16, _blk, 0)

    def _drain(r, _):
        pltpu.make_async_copy(
            table_hbm.at[pl.ds(1, MAX_LEN)], tbl_v, sem).wait()
        return 0

    lax.fori_loop(0, RPW, _drain, 0)

    pltpu.sync_copy(pos_v, pos_hbm.at[pl.ds(base * MAX_LEN, RPW * MAX_LEN)])


def kernel(input_len, table):
    emb, pos_flat = _pe_kernel(input_len, table)
    return emb, pos_flat.reshape(BATCH, MAX_LEN)


# speedup vs baseline: 1.0299x; 1.0299x over previous
import functools

import jax
import jax.numpy as jnp
from jax import lax
from jax.experimental import pallas as pl
from jax.experimental.pallas import tpu as pltpu
from jax.experimental.pallas import tpu_sc as plsc

MODEL_DIM = 128
MAX_LEN = 200
BATCH = 4096

NC = 2
NS = 16
NW = NC * NS
RPW = BATCH // NW
ZROWS = 128

_mesh = plsc.VectorSubcoreMesh(
    core_axis_name="c", subcore_axis_name="s", num_cores=NC, num_subcores=NS)


@functools.partial(
    pl.kernel,
    out_type=(
        jax.ShapeDtypeStruct((BATCH, MAX_LEN, MODEL_DIM), jnp.float32),
        jax.ShapeDtypeStruct((BATCH * MAX_LEN,), jnp.int32),
    ),
    mesh=_mesh,
    compiler_params=pltpu.CompilerParams(use_tc_tiling_on_sc=False),
    scratch_types=[
        pltpu.VMEM((MAX_LEN, MODEL_DIM), jnp.float32),
        pltpu.VMEM((ZROWS, MODEL_DIM), jnp.float32),
        pltpu.VMEM((RPW,), jnp.int32),
        pltpu.VMEM((RPW * MAX_LEN,), jnp.int32),
        pltpu.SemaphoreType.DMA,
    ],
)
def _pe_kernel(len_hbm, table_hbm, emb_hbm, pos_hbm,
               tbl_v, zero_v, lens_v, pos_v, sem):
    wid = lax.axis_index("s") * NC + lax.axis_index("c")
    base = wid * RPW

    pltpu.sync_copy(table_hbm.at[pl.ds(1, MAX_LEN)], tbl_v)
    pltpu.sync_copy(len_hbm.at[pl.ds(base, RPW)], lens_v)

    zvec = jnp.zeros((16,), jnp.float32)

    def _zero_row(r, _):
        for c in range(MODEL_DIM // 16):
            zero_v[r, pl.ds(c * 16, 16)] = zvec
        return 0

    lax.fori_loop(0, ZROWS, _zero_row, 0)

    iota = lax.iota(jnp.int32, 16)

    def _copy_digits(from_table, row, start, n):
        for b in (128, 64, 32, 16, 8, 4, 2, 1):
            rel = n & ~(2 * b - 1)

            @pl.when((n & b) != 0)
            def _():
                if from_table:
                    src = tbl_v.at[pl.ds(start + rel, b)]
                else:
                    src = zero_v.at[pl.ds(0, b)]
                pltpu.async_copy(
                    src, emb_hbm.at[row, pl.ds(start + rel, b)], sem)

    def _blk(blk, _):
        lv = lens_v[pl.ds(blk * 16, 16)]
        for lane in range(16):
            L = lv[lane]
            r = blk * 16 + lane
            row = base + r
            _copy_digits(True, row, 0, L)
            _copy_digits(False, row, L, MAX_LEN - L)
        for p in range(8):
            la = lv[2 * p]
            lb = lv[2 * p + 1]
            sa = jnp.full((16,), la, jnp.int32)
            sb = jnp.full((16,), lb, jnp.int32)
            fbase = (blk * 16 + 2 * p) * MAX_LEN
            for c in range(25):
                if c <= 11:
                    vec = iota + (16 * c + 1)
                    splat = sa
                elif c == 12:
                    vec = jnp.where(iota < 8, iota + 193, iota - 7)
                    splat = jnp.where(iota < 8, sa, sb)
                else:
                    vec = iota + (16 * c - 200 + 1)
                    splat = sb
                pos_v[pl.ds(fbase + 16 * c, 16)] = jnp.where(
                    vec <= splat, vec, 0)
        return 0

    lax.fori_loop(0, RPW // 16, _blk, 0)

    def _drain(r, _):
        pltpu.make_async_copy(
            table_hbm.at[pl.ds(1, MAX_LEN)], tbl_v, sem).wait()
        return 0

    lax.fori_loop(0, RPW, _drain, 0)

    pltpu.sync_copy(pos_v, pos_hbm.at[pl.ds(base * MAX_LEN, RPW * MAX_LEN)])


def kernel(input_len, table):
    emb, pos_flat = _pe_kernel(input_len, table)
    return emb, pos_flat.reshape(BATCH, MAX_LEN)
